# initial kernel scaffold (unmeasured)
import jax
import jax.numpy as jnp
from jax import lax
from jax.experimental import pallas as pl
from jax.experimental.pallas import tpu as pltpu


def kernel(
    x,
):
    def body(*refs):
        pass

    out_shape = jax.ShapeDtypeStruct(..., jnp.float32)
    return pl.pallas_call(body, out_shape=out_shape)(...)


# baseline (device time: 145924 ns/iter reference)
import jax
import jax.numpy as jnp
from jax import lax
from jax.experimental import pallas as pl
from jax.experimental.pallas import tpu as pltpu

K = 32
ROW_CHUNK = 128


def _topk_desc(w, k):
    outs = []
    for _ in range(k):
        m = jnp.max(w, axis=1, keepdims=True)
        outs.append(m)
        w = jnp.where(w == m, -jnp.inf, w)
    return jnp.concatenate(outs, axis=1)


def kernel(x):
    m_rows, n_loc = x.shape
    n_chunks = m_rows // ROW_CHUNK

    def body(x_hbm, o_ref, work_ref, cand_ref, peer_ref,
             load_sem, send_sem, recv_sem):
        my_x = lax.axis_index("x")
        my_y = lax.axis_index("y")
        my_z = lax.axis_index("z")
        peer = (my_x, my_y, 1 - my_z)

        barrier_sem = pltpu.get_barrier_semaphore()
        pl.semaphore_signal(
            barrier_sem, inc=1,
            device_id=peer, device_id_type=pl.DeviceIdType.MESH,
        )
        pl.semaphore_wait(barrier_sem, 1)

        def chunk_body(c, carry):
            copy = pltpu.make_async_copy(
                x_hbm.at[pl.ds(c * ROW_CHUNK, ROW_CHUNK), :],
                work_ref,
                load_sem,
            )
            copy.start()
            copy.wait()
            cand_ref[pl.ds(c * ROW_CHUNK, ROW_CHUNK), :] = _topk_desc(
                work_ref[...], K
            )
            return carry

        lax.fori_loop(0, n_chunks, chunk_body, 0)

        rdma = pltpu.make_async_remote_copy(
            src_ref=cand_ref,
            dst_ref=peer_ref,
            send_sem=send_sem,
            recv_sem=recv_sem,
            device_id=peer,
            device_id_type=pl.DeviceIdType.MESH,
        )
        rdma.start()
        rdma.wait()

        merged = jnp.concatenate([cand_ref[...], peer_ref[...]], axis=1)
        o_ref[...] = _topk_desc(merged, K)

    return pl.pallas_call(
        body,
        out_shape=jax.ShapeDtypeStruct((m_rows, K), jnp.float32),
        in_specs=[pl.BlockSpec(memory_space=pl.ANY)],
        out_specs=pl.BlockSpec(memory_space=pltpu.VMEM),
        scratch_shapes=[
            pltpu.VMEM((ROW_CHUNK, n_loc), jnp.float32),
            pltpu.VMEM((m_rows, K), jnp.float32),
            pltpu.VMEM((m_rows, K), jnp.float32),
            pltpu.SemaphoreType.DMA,
            pltpu.SemaphoreType.DMA,
            pltpu.SemaphoreType.DMA,
        ],
        compiler_params=pltpu.CompilerParams(collective_id=0),
    )(x)


# device time: 50570 ns/iter; 2.8856x vs baseline; 2.8856x over previous
import jax
import jax.numpy as jnp
from jax import lax
from jax.experimental import pallas as pl
from jax.experimental.pallas import tpu as pltpu

K = 32
ROW_CHUNK = 128
N_XY = 4


def _topk_desc(w, k):
    outs = []
    for _ in range(k):
        m = jnp.max(w, axis=1, keepdims=True)
        outs.append(m)
        w = jnp.where(w == m, -jnp.inf, w)
    return jnp.concatenate(outs, axis=1)


def kernel(x):
    m_rows, n_loc = x.shape
    blk = m_rows // N_XY
    n_chunks = blk // ROW_CHUNK

    def body(x_hbm, o_ref, work_ref, cand_ref, peer_ref, load_sem,
             zs_sem, zr_sem, xs_sem, xr_sem,
             ys0_sem, yr0_sem, ys1_sem, yr1_sem):
        my_x = lax.axis_index("x")
        my_y = lax.axis_index("y")
        my_z = lax.axis_index("z")
        zpeer = (my_x, my_y, 1 - my_z)
        xpeer = (1 - my_x, my_y, my_z)
        ypeer = (my_x, 1 - my_y, my_z)

        barrier_sem = pltpu.get_barrier_semaphore()
        for p in (zpeer, xpeer, ypeer):
            pl.semaphore_signal(
                barrier_sem, inc=1,
                device_id=p, device_id_type=pl.DeviceIdType.MESH,
            )
        pl.semaphore_wait(barrier_sem, 3)

        b = my_x * 2 + my_y
        row0 = b * blk

        def chunk_body(c, carry):
            copy = pltpu.make_async_copy(
                x_hbm.at[pl.ds(row0 + c * ROW_CHUNK, ROW_CHUNK), :],
                work_ref,
                load_sem,
            )
            copy.start()
            copy.wait()
            cand_ref[pl.ds(c * ROW_CHUNK, ROW_CHUNK), :] = _topk_desc(
                work_ref[...], K
            )
            return carry

        lax.fori_loop(0, n_chunks, chunk_body, 0)

        zrdma = pltpu.make_async_remote_copy(
            src_ref=cand_ref, dst_ref=peer_ref,
            send_sem=zs_sem, recv_sem=zr_sem,
            device_id=zpeer, device_id_type=pl.DeviceIdType.MESH,
        )
        zrdma.start()
        zrdma.wait()
        merged = jnp.concatenate([cand_ref[...], peer_ref[...]], axis=1)
        o_ref[pl.ds(row0, blk), :] = _topk_desc(merged, K)

        xrdma = pltpu.make_async_remote_copy(
            src_ref=o_ref.at[pl.ds(row0, blk), :],
            dst_ref=o_ref.at[pl.ds(row0, blk), :],
            send_sem=xs_sem, recv_sem=xr_sem,
            device_id=xpeer, device_id_type=pl.DeviceIdType.MESH,
        )
        xrdma.start()
        xrdma.wait()

        bx = (1 - my_x) * 2 + my_y
        rowx = bx * blk
        y0 = pltpu.make_async_remote_copy(
            src_ref=o_ref.at[pl.ds(row0, blk), :],
            dst_ref=o_ref.at[pl.ds(row0, blk), :],
            send_sem=ys0_sem, recv_sem=yr0_sem,
            device_id=ypeer, device_id_type=pl.DeviceIdType.MESH,
        )
        y1 = pltpu.make_async_remote_copy(
            src_ref=o_ref.at[pl.ds(rowx, blk), :],
            dst_ref=o_ref.at[pl.ds(rowx, blk), :],
            send_sem=ys1_sem, recv_sem=yr1_sem,
            device_id=ypeer, device_id_type=pl.DeviceIdType.MESH,
        )
        y0.start()
        y1.start()
        y0.wait()
        y1.wait()

    return pl.pallas_call(
        body,
        out_shape=jax.ShapeDtypeStruct((m_rows, K), jnp.float32),
        in_specs=[pl.BlockSpec(memory_space=pl.ANY)],
        out_specs=pl.BlockSpec(memory_space=pltpu.VMEM),
        scratch_shapes=[
            pltpu.VMEM((ROW_CHUNK, n_loc), jnp.float32),
            pltpu.VMEM((blk, K), jnp.float32),
            pltpu.VMEM((blk, K), jnp.float32),
            pltpu.SemaphoreType.DMA,
            pltpu.SemaphoreType.DMA,
            pltpu.SemaphoreType.DMA,
            pltpu.SemaphoreType.DMA,
            pltpu.SemaphoreType.DMA,
            pltpu.SemaphoreType.DMA,
            pltpu.SemaphoreType.DMA,
            pltpu.SemaphoreType.DMA,
            pltpu.SemaphoreType.DMA,
        ],
        compiler_params=pltpu.CompilerParams(collective_id=0),
    )(x)


# device time: 36226 ns/iter; 4.0282x vs baseline; 1.3960x over previous
import jax
import jax.numpy as jnp
from jax import lax
from jax.experimental import pallas as pl
from jax.experimental.pallas import tpu as pltpu

K = 32
ROW_CHUNK = 128
N_XY = 4
STOP_W = 8


def _bitonic_sort_desc(arrs):
    n = len(arrs)
    k = 2
    while k <= n:
        j = k // 2
        while j >= 1:
            for i in range(n):
                l = i ^ j
                if l > i:
                    hi = jnp.maximum(arrs[i], arrs[l])
                    lo = jnp.minimum(arrs[i], arrs[l])
                    if (i & k) == 0:
                        arrs[i], arrs[l] = hi, lo
                    else:
                        arrs[i], arrs[l] = lo, hi
            j //= 2
        k *= 2
    return arrs


def _bitonic_merge_desc(arrs):
    n = len(arrs)
    j = n // 2
    while j >= 1:
        for i in range(n):
            l = i ^ j
            if l > i:
                hi = jnp.maximum(arrs[i], arrs[l])
                lo = jnp.minimum(arrs[i], arrs[l])
                arrs[i], arrs[l] = hi, lo
        j //= 2
    return arrs


def _topk_desc(w, k):
    outs = []
    for _ in range(k):
        m = jnp.max(w, axis=1, keepdims=True)
        outs.append(m)
        w = jnp.where(w == m, -jnp.inf, w)
    return jnp.concatenate(outs, axis=1)


def _topk_bitonic(w, k):
    _, c = w.shape
    g = c // k
    s = [w[:, i * g:(i + 1) * g] for i in range(k)]
    s = _bitonic_sort_desc(s)
    width = g
    while width > STOP_W:
        half = width // 2
        m = [jnp.maximum(s[i][:, :half], s[k - 1 - i][:, half:])
             for i in range(k)]
        s = _bitonic_merge_desc(m)
        width = half
    return _topk_desc(jnp.concatenate(s, axis=1), k)


def kernel(x):
    m_rows, n_loc = x.shape
    blk = m_rows // N_XY
    n_chunks = blk // ROW_CHUNK

    def body(x_hbm, o_ref, work_ref, cand_ref, peer_ref,
             load_sems, zs_sem, zr_sem, gs_sems, gr_sems):
        my_x = lax.axis_index("x")
        my_y = lax.axis_index("y")
        my_z = lax.axis_index("z")
        zpeer = (my_x, my_y, 1 - my_z)
        gpeers = (
            (1 - my_x, my_y, my_z),
            (my_x, 1 - my_y, my_z),
            (1 - my_x, 1 - my_y, my_z),
        )

        barrier_sem = pltpu.get_barrier_semaphore()
        for p in (zpeer,) + gpeers:
            pl.semaphore_signal(
                barrier_sem, inc=1,
                device_id=p, device_id_type=pl.DeviceIdType.MESH,
            )
        pl.semaphore_wait(barrier_sem, 4)

        b = my_x * 2 + my_y
        row0 = b * blk

        def load(c):
            return pltpu.make_async_copy(
                x_hbm.at[pl.ds(row0 + c * ROW_CHUNK, ROW_CHUNK), :],
                work_ref.at[c],
                load_sems.at[c],
            )

        for c in range(n_chunks):
            load(c).start()
        for c in range(n_chunks):
            load(c).wait()
            cand_ref[pl.ds(c * ROW_CHUNK, ROW_CHUNK), :] = _topk_bitonic(
                work_ref[c], K
            )

        zrdma = pltpu.make_async_remote_copy(
            src_ref=cand_ref, dst_ref=peer_ref,
            send_sem=zs_sem, recv_sem=zr_sem,
            device_id=zpeer, device_id_type=pl.DeviceIdType.MESH,
        )
        zrdma.start()
        zrdma.wait()
        merged = jnp.concatenate([cand_ref[...], peer_ref[...]], axis=1)
        o_ref[pl.ds(row0, blk), :] = _topk_desc(merged, K)

        rdmas = []
        for i, p in enumerate(gpeers):
            rdmas.append(pltpu.make_async_remote_copy(
                src_ref=o_ref.at[pl.ds(row0, blk), :],
                dst_ref=o_ref.at[pl.ds(row0, blk), :],
                send_sem=gs_sems.at[i], recv_sem=gr_sems.at[i],
                device_id=p, device_id_type=pl.DeviceIdType.MESH,
            ))
        for r in rdmas:
            r.start()
        for r in rdmas:
            r.wait()

    return pl.pallas_call(
        body,
        out_shape=jax.ShapeDtypeStruct((m_rows, K), jnp.float32),
        in_specs=[pl.BlockSpec(memory_space=pl.ANY)],
        out_specs=pl.BlockSpec(memory_space=pltpu.VMEM),
        scratch_shapes=[
            pltpu.VMEM((n_chunks, ROW_CHUNK, n_loc), jnp.float32),
            pltpu.VMEM((blk, K), jnp.float32),
            pltpu.VMEM((blk, K), jnp.float32),
            pltpu.SemaphoreType.DMA((2,)),
            pltpu.SemaphoreType.DMA,
            pltpu.SemaphoreType.DMA,
            pltpu.SemaphoreType.DMA((3,)),
            pltpu.SemaphoreType.DMA((3,)),
        ],
        compiler_params=pltpu.CompilerParams(collective_id=0),
    )(x)
